# Initial kernel scaffold; baseline (speedup 1.0000x reference)
import jax
import jax.numpy as jnp
from jax.experimental import pallas as pl


def kernel(cats, nums, emb_tables, bn_num_g, bn_num_b, W1, b1, bn1_g, bn1_b, W2, b2, bn2_g, bn2_b, W3, b3):
    raise NotImplementedError("write your pallas kernel here")


# speedup vs baseline: 1.4292x; 1.4292x over previous
import functools

import jax
import jax.numpy as jnp
from jax import lax
from jax.experimental import pallas as pl
from jax.experimental.pallas import tpu as pltpu
from jax.experimental.pallas import tpu_sc as plsc

B = 4096
F = 26
V = 100000
E = 50
NUM = 13
H1 = 512
H2 = 256

N = B * F
NW = 32
PER_W = N // NW
GE = 256
NG = PER_W // GE
WROWS = N * E // 64 + 1
TROWS = F * V * E // 64


@jax.jit
def _gather_sc(table64, widx, off):
    mesh = plsc.VectorSubcoreMesh(core_axis_name="core", subcore_axis_name="subcore")

    @functools.partial(
        pl.kernel,
        out_type=jax.ShapeDtypeStruct((N * E,), jnp.float32),
        mesh=mesh,
        scratch_types=[
            pltpu.VMEM((2 * PER_W,), jnp.int32),
            pltpu.VMEM((PER_W,), jnp.int32),
            pltpu.VMEM((2 * GE, 64), jnp.float32),
            pltpu.VMEM((GE * E,), jnp.float32),
            pltpu.SemaphoreType.DMA,
        ],
        compiler_params=pltpu.CompilerParams(
            use_tc_tiling_on_sc=False, needs_layout_passes=False),
    )
    def k(table_hbm, widx_hbm, off_hbm, out_hbm, widx_v, off_v, win_v, out_v, sem):
        wid = lax.axis_index("subcore") * 2 + lax.axis_index("core")
        pltpu.sync_copy(widx_hbm.at[wid], widx_v)
        pltpu.sync_copy(off_hbm.at[wid], off_v)
        lane = lax.iota(jnp.int32, 16)
        base_r0 = lane * 2
        base_d0 = lane * E

        @pl.loop(0, NG)
        def _g(gi):
            pltpu.async_copy(
                table_hbm.at[widx_v.at[pl.ds(gi * (2 * GE), 2 * GE)]],
                win_v, sem).wait()

            @pl.loop(0, GE // 16)
            def _grp(pi):
                o = off_v[pl.ds(gi * GE + pi * 16, 16)]
                base_r = base_r0 + pi * 32
                base_d = base_d0 + pi * (16 * E)
                for j in range(E):
                    w = o + j
                    r = base_r + (w >> 6)
                    c = w & 63
                    x = plsc.load_gather(win_v, [r, c])
                    plsc.store_scatter(out_v, [base_d + j], x)

            pltpu.sync_copy(
                out_v,
                out_hbm.at[pl.ds(wid * (PER_W * E) + gi * (GE * E), GE * E)])

    return k(table64, widx, off)


def _bn(x, g, b, eps=1e-5):
    m = jnp.mean(x, axis=0, keepdims=True)
    v = jnp.mean((x - m) * (x - m), axis=0, keepdims=True)
    return (x - m) * lax.rsqrt(v + eps) * g + b


def _mlp_body(xc_ref, xn_ref, w1c_ref, w1n_ref, b1_ref, g1_ref, bb1_ref,
              w2_ref, b2_ref, g2_ref, bb2_ref, w3_ref, b3_ref,
              gn_ref, bnb_ref, out_ref):
    x_num = _bn(xn_ref[...], gn_ref[...], bnb_ref[...])
    h = jnp.dot(xc_ref[...], w1c_ref[...], preferred_element_type=jnp.float32)
    h = h + jnp.dot(x_num, w1n_ref[...], preferred_element_type=jnp.float32)
    h = h + b1_ref[...]
    h = jnp.maximum(_bn(h, g1_ref[...], bb1_ref[...]), 0.0)
    h = jnp.dot(h, w2_ref[...], preferred_element_type=jnp.float32) + b2_ref[...]
    h = jnp.maximum(_bn(h, g2_ref[...], bb2_ref[...]), 0.0)
    out_ref[...] = jnp.dot(h, w3_ref[...], preferred_element_type=jnp.float32) + b3_ref[...]


@jax.jit
def _mlp_tc(xc, xn, w1c, w1n, b1, g1, bb1, w2, b2, g2, bb2, w3, b3, gn, bnb):
    return pl.pallas_call(
        _mlp_body,
        out_shape=jax.ShapeDtypeStruct((B, 1), jnp.float32),
    )(xc, xn, w1c, w1n, b1, g1, bb1, w2, b2, g2, bb2, w3, b3, gn, bnb)


def kernel(cats, nums, emb_tables, bn_num_g, bn_num_b, W1, b1, bn1_g, bn1_b,
           W2, b2, bn2_g, bn2_b, W3, b3):
    table64 = emb_tables.reshape(TROWS, 64)
    idx = (cats.astype(jnp.int32)
           + (jnp.arange(F, dtype=jnp.int32) * V)[None, :]).reshape(N)
    word = idx * E
    w0 = word >> 6
    off = (word & 63).reshape(NW, PER_W)
    w1_ = jnp.minimum(w0 + 1, TROWS - 1)
    widx = jnp.stack([w0, w1_], axis=1).reshape(NW, 2 * PER_W)
    rows = _gather_sc(table64, widx, off)
    xcat = rows.reshape(B, F * E)

    w1c = W1[:, : F * E].T
    w1n = W1[:, F * E:].T
    out = _mlp_tc(
        xcat, nums,
        w1c, w1n, b1[None, :], bn1_g[None, :], bn1_b[None, :],
        W2.T, b2[None, :], bn2_g[None, :], bn2_b[None, :],
        W3.T, b3[None, :],
        bn_num_g[None, :], bn_num_b[None, :],
    )
    return out[:, 0]
